# Initial kernel scaffold; baseline (speedup 1.0000x reference)
import jax
import jax.numpy as jnp
from jax.experimental import pallas as pl


def kernel(z_grid, mt, W, b):
    raise NotImplementedError("write your pallas kernel here")


# speedup vs baseline: 13.5533x; 13.5533x over previous
import functools

import jax
import jax.numpy as jnp
from jax import lax
from jax.experimental import pallas as pl
from jax.experimental.pallas import tpu as pltpu
from jax.experimental.pallas import tpu_sc as plsc

M, G, DZ = 16, 16384, 128
NT, DY = 8192, 64
B = M * NT

NC, NS = 2, 16
NW = NC * NS
B_PER_W = B // NW
CH = 128
NCHUNK = B_PER_W // CH


def _sc_gather():
    mesh = plsc.VectorSubcoreMesh(core_axis_name="c", subcore_axis_name="s")

    @functools.partial(
        pl.kernel,
        mesh=mesh,
        out_type=jax.ShapeDtypeStruct((B, DZ), jnp.float32),
        scratch_types=[
            pltpu.VMEM((NCHUNK, CH), jnp.int32),
            pltpu.VMEM((CH, DZ), jnp.float32),
            pltpu.VMEM((CH, DZ), jnp.float32),
            pltpu.SemaphoreType.DMA,
            pltpu.SemaphoreType.DMA,
            pltpu.SemaphoreType.DMA,
            pltpu.SemaphoreType.DMA,
        ],
    )
    def gather(table_hbm, idx_hbm, out_hbm, idx_v, rows0, rows1, g0, g1, s0, s1):
        wid = lax.axis_index("s") * NC + lax.axis_index("c")
        base = wid * B_PER_W
        pltpu.sync_copy(idx_hbm.at[wid], idx_v)

        rows = (rows0, rows1)
        gsem = (g0, g1)
        ssem = (s0, s1)
        gcp = [None, None]
        scp = [None, None]
        gcp[0] = pltpu.async_copy(table_hbm.at[idx_v.at[0]], rows[0], gsem[0])
        for c in range(NCHUNK):
            b = c & 1
            nb = (c + 1) & 1
            if c + 1 < NCHUNK:
                if scp[nb] is not None:
                    scp[nb].wait()
                gcp[nb] = pltpu.async_copy(
                    table_hbm.at[idx_v.at[c + 1]], rows[nb], gsem[nb]
                )
            gcp[b].wait()
            scp[b] = pltpu.async_copy(
                rows[b], out_hbm.at[pl.ds(base + c * CH, CH)], ssem[b]
            )
        for b in range(2):
            if scp[b] is not None:
                scp[b].wait()

    return gather


_gather_fn = _sc_gather()


def _mm_body(zt_ref, w_ref, b_ref, o_ref):
    o_ref[...] = (
        jnp.dot(zt_ref[...], w_ref[...], preferred_element_type=jnp.float32)
        + b_ref[...]
    )


def _tc_linear(zt, W, b2):
    BM = 2048
    return pl.pallas_call(
        _mm_body,
        grid=(B // BM,),
        in_specs=[
            pl.BlockSpec((BM, DZ), lambda i: (i, 0)),
            pl.BlockSpec((DZ, DY), lambda i: (0, 0)),
            pl.BlockSpec((1, DY), lambda i: (0, 0)),
        ],
        out_specs=pl.BlockSpec((BM, DY), lambda i: (i, 0)),
        out_shape=jax.ShapeDtypeStruct((B, DY), jnp.float32),
    )(zt, W, b2)


@jax.jit
def kernel(z_grid, mt, W, b):
    table = z_grid.reshape(M * G, DZ)
    offs = (jnp.arange(M, dtype=jnp.int32) * G)[:, None]
    flat_idx = (mt.astype(jnp.int32) + offs).reshape(NW, NCHUNK, CH)
    zt = _gather_fn(table, flat_idx)
    out = _tc_linear(zt, W, b.reshape(1, DY))
    return out.reshape(M, NT, DY)
